# Initial kernel scaffold; baseline (speedup 1.0000x reference)
import jax
import jax.numpy as jnp
from jax.experimental import pallas as pl


def kernel(x, edge_index, W1, b1, W2, b2, Wn, Ws):
    raise NotImplementedError("write your pallas kernel here")


# speedup vs baseline: 10.3946x; 10.3946x over previous
import functools
import jax
import jax.numpy as jnp
from jax import lax
from jax.experimental import pallas as pl
from jax.experimental.pallas import tpu as pltpu
from jax.experimental.pallas import tpu_sc as plsc

N = 10000
E = 320000
D_IN = 128
HID = 128

NC = 2
NS = 16
NW = NC * NS
K = 80
NPAD = 10240
DEGW = 128

@functools.cache
def _mesh():
  return plsc.VectorSubcoreMesh(
      core_axis_name="c", subcore_axis_name="s", num_cores=NC, num_subcores=NS
  )


def _zero_vmem2d(buf, rows, cols):
  zv = jnp.zeros((16,), jnp.float32)

  @pl.loop(0, rows)
  def _(r):
    for k in range(cols // 16):
      buf[r, pl.ds(k * 16, 16)] = zv


@functools.cache
def _deg_kernel():
  return pl.kernel(
      _deg_body,
      out_type=jax.ShapeDtypeStruct((NC, NPAD, DEGW), jnp.float32),
      mesh=_mesh(),
      scratch_types=[
          pltpu.VMEM((K,), jnp.int32),
          pltpu.VMEM((K, DEGW), jnp.float32),
          pltpu.VMEM((128, DEGW), jnp.float32),
          pltpu.VMEM_SHARED((NPAD, DEGW), jnp.float32),
      ],
  )


def _deg_body(dst_hbm, deg_hbm, didx, ones, zslab, deg_sh):
  c = lax.axis_index("c")
  s = lax.axis_index("s")

  ov = jnp.full((16,), 1.0, jnp.float32)

  @pl.loop(0, K)
  def _(r):
    for k in range(DEGW // 16):
      ones[r, pl.ds(k * 16, 16)] = ov

  _zero_vmem2d(zslab, 128, DEGW)
  rows_per_tile = NPAD // NS
  for t in range(rows_per_tile // 128):
    pltpu.sync_copy(zslab, deg_sh.at[pl.ds(s * rows_per_tile + t * 128, 128)])
  plsc.subcore_barrier()

  per_worker = E // NW
  base = c * (E // NC) + s * per_worker

  @pl.loop(0, per_worker // K)
  def _(j):
    pltpu.sync_copy(dst_hbm.at[pl.ds(base + j * K, K)], didx)
    pltpu.sync_copy(ones, deg_sh.at[didx], add=True)

  plsc.subcore_barrier()
  pltpu.sync_copy(
      deg_sh.at[pl.ds(s * rows_per_tile, rows_per_tile)],
      deg_hbm.at[c, pl.ds(s * rows_per_tile, rows_per_tile)],
  )


def _edge_pass(src_hbm, dst_hbm, y_ref, sidx, didx, rows, sem, z_sh, base, n_chunks):
  @pl.loop(0, n_chunks)
  def _(j):
    off = base + j * K
    pltpu.sync_copy(src_hbm.at[pl.ds(off, K)], sidx)
    pltpu.sync_copy(dst_hbm.at[pl.ds(off, K)], didx)
    pltpu.async_copy(y_ref.at[sidx], rows, sem).wait()
    pltpu.sync_copy(rows, z_sh.at[didx], add=True)


def _scatter_scratch():
  return [
      pltpu.VMEM((K,), jnp.int32),
      pltpu.VMEM((K,), jnp.int32),
      pltpu.VMEM((K, HID), jnp.float32),
      pltpu.VMEM((128, HID), jnp.float32),
      pltpu.VMEM_SHARED((NPAD, HID), jnp.float32),
      pltpu.SemaphoreType.DMA,
  ]


def _zero_and_barrier(zslab, z_sh, s):
  _zero_vmem2d(zslab, 128, HID)
  rows_per_tile = NPAD // NS
  for t in range(rows_per_tile // 128):
    pltpu.sync_copy(zslab, z_sh.at[pl.ds(s * rows_per_tile + t * 128, 128)])
  plsc.subcore_barrier()


def _copy_out(z_sh, z_hbm, c, s):
  rows_out = NPAD // NS
  pltpu.sync_copy(
      z_sh.at[pl.ds(s * rows_out, rows_out)],
      z_hbm.at[c, pl.ds(s * rows_out, rows_out)],
  )


@functools.cache
def _scatter_l1():
  return pl.kernel(
      _scatter_l1_body,
      out_type=jax.ShapeDtypeStruct((NC, NPAD, HID), jnp.float32),
      mesh=_mesh(),
      scratch_types=_scatter_scratch(),
  )


def _scatter_l1_body(ya_hbm, yb_hbm, src_hbm, dst_hbm, z_hbm, sidx, didx, rows, zslab, z_sh, sem):
  c = lax.axis_index("c")
  s = lax.axis_index("s")
  _zero_and_barrier(zslab, z_sh, s)

  per_tile = E // NS
  base = s * per_tile
  n_chunks = per_tile // K

  @pl.when(c == 0)
  def _():
    _edge_pass(src_hbm, dst_hbm, ya_hbm, sidx, didx, rows, sem, z_sh, base, n_chunks)

  @pl.when(c == 1)
  def _():
    _edge_pass(src_hbm, dst_hbm, yb_hbm, sidx, didx, rows, sem, z_sh, base, n_chunks)

  plsc.subcore_barrier()
  _copy_out(z_sh, z_hbm, c, s)


@functools.cache
def _scatter_l2():
  return pl.kernel(
      _scatter_l2_body,
      out_type=jax.ShapeDtypeStruct((NC, NPAD, HID), jnp.float32),
      mesh=_mesh(),
      scratch_types=_scatter_scratch(),
  )


def _scatter_l2_body(y_hbm, src_hbm, dst_hbm, z_hbm, sidx, didx, rows, zslab, z_sh, sem):
  c = lax.axis_index("c")
  s = lax.axis_index("s")
  _zero_and_barrier(zslab, z_sh, s)

  per_worker = E // NW
  base = c * (E // NC) + s * per_worker
  _edge_pass(src_hbm, dst_hbm, y_hbm, sidx, didx, rows, sem, z_sh, base, per_worker // K)

  plsc.subcore_barrier()
  _copy_out(z_sh, z_hbm, c, s)


RB = 1000
GRID = N // RB


def _tc1_body(x_ref, w_ref, xw1_ref, hn_ref, hs_ref, hg_ref):
  i = pl.program_id(0)
  xw = jnp.dot(x_ref[...], w_ref[...], preferred_element_type=jnp.float32)
  xw1_ref[...] = xw[:, : 2 * HID]
  hn = xw[:, 2 * HID : 3 * HID]
  hn_ref[...] = hn
  hs_ref[...] = xw[:, 3 * HID :]

  @pl.when(i == 0)
  def _():
    hg_ref[...] = jnp.zeros_like(hg_ref)

  hg_ref[...] += jnp.sum(hn, axis=0, keepdims=True) * (1.0 / N)


def _tc2_body(xw1_ref, dega_ref, degb_ref, ya_ref, yb_ref, dinv_ref):
  deg = dega_ref[0][:, 0:1] + degb_ref[0][:, 0:1] + 1.0
  dinv = lax.rsqrt(deg)
  dinv_ref[...] = dinv
  y = xw1_ref[...] * dinv
  ya_ref[...] = y[:, :HID]
  yb_ref[...] = y[:, HID:]


def _tc3_body(za_ref, zb_ref, ya_ref, yb_ref, dinv_ref, b1_ref, w2_ref, y2_ref):
  dinv = dinv_ref[...]
  h = jnp.concatenate(
      [za_ref[0] + ya_ref[...], zb_ref[0] + yb_ref[...]], axis=1
  ) * dinv + b1_ref[...]
  h = jnp.maximum(h, 0.0)
  y2_ref[...] = jnp.dot(h, w2_ref[...], preferred_element_type=jnp.float32) * dinv


def _tc4_body(za_ref, zb_ref, y2_ref, dinv_ref, b2_ref, out_ref):
  v = (za_ref[0] + zb_ref[0] + y2_ref[...]) * dinv_ref[...] + b2_ref[...]
  out_ref[...] = jnp.maximum(v, 0.0)


def _row_spec(width):
  return pl.BlockSpec((RB, width), lambda i: (i, 0))


def _part_spec(width):
  def mk(p):
    return pl.BlockSpec((1, RB, width), lambda i, p=p: (p, i, 0))
  return mk


def kernel(x, edge_index, W1, b1, W2, b2, Wn, Ws):
  wcat = jnp.concatenate([W1, Wn, Ws], axis=1)
  src = edge_index[0]
  dst = edge_index[1]

  xw1, h_node, h_sub, h_graph = pl.pallas_call(
      _tc1_body,
      grid=(GRID,),
      in_specs=[
          _row_spec(D_IN),
          pl.BlockSpec((D_IN, 4 * HID), lambda i: (0, 0)),
      ],
      out_specs=[
          _row_spec(2 * HID),
          _row_spec(HID),
          _row_spec(HID),
          pl.BlockSpec((1, HID), lambda i: (0, 0)),
      ],
      out_shape=[
          jax.ShapeDtypeStruct((N, 2 * HID), jnp.float32),
          jax.ShapeDtypeStruct((N, HID), jnp.float32),
          jax.ShapeDtypeStruct((N, HID), jnp.float32),
          jax.ShapeDtypeStruct((1, HID), jnp.float32),
      ],
      compiler_params=pltpu.CompilerParams(
          dimension_semantics=("arbitrary",)
      ),
  )(x, wcat)

  degp = _deg_kernel()(dst)

  dspec = _part_spec(DEGW)
  y1a, y1b, dinv = pl.pallas_call(
      _tc2_body,
      grid=(GRID,),
      in_specs=[_row_spec(2 * HID), dspec(0), dspec(1)],
      out_specs=[_row_spec(HID), _row_spec(HID), pl.BlockSpec((RB, 1), lambda i: (i, 0))],
      out_shape=[
          jax.ShapeDtypeStruct((N, HID), jnp.float32),
          jax.ShapeDtypeStruct((N, HID), jnp.float32),
          jax.ShapeDtypeStruct((N, 1), jnp.float32),
      ],
  )(xw1, degp, degp)

  z1 = _scatter_l1()(y1a, y1b, src, dst)

  zspec = _part_spec(HID)
  y2 = pl.pallas_call(
      _tc3_body,
      grid=(GRID,),
      in_specs=[
          zspec(0),
          zspec(1),
          _row_spec(HID),
          _row_spec(HID),
          pl.BlockSpec((RB, 1), lambda i: (i, 0)),
          pl.BlockSpec((1, 2 * HID), lambda i: (0, 0)),
          pl.BlockSpec((2 * HID, HID), lambda i: (0, 0)),
      ],
      out_specs=_row_spec(HID),
      out_shape=jax.ShapeDtypeStruct((N, HID), jnp.float32),
  )(z1, z1, y1a, y1b, dinv, b1.reshape(1, 2 * HID), W2)

  z2 = _scatter_l2()(y2, src, dst)

  h_gnn = pl.pallas_call(
      _tc4_body,
      grid=(GRID,),
      in_specs=[
          zspec(0),
          zspec(1),
          _row_spec(HID),
          pl.BlockSpec((RB, 1), lambda i: (i, 0)),
          pl.BlockSpec((1, HID), lambda i: (0, 0)),
      ],
      out_specs=_row_spec(HID),
      out_shape=jax.ShapeDtypeStruct((N, HID), jnp.float32),
  )(z2, z2, y2, dinv, b2.reshape(1, HID))

  return (h_gnn, h_node, h_sub, h_graph)
